# baseline (speedup 1.0000x reference)
import functools

import jax
import jax.numpy as jnp
import numpy as np
from jax import lax
from jax.experimental import pallas as pl
from jax.experimental.pallas import tpu as pltpu
from jax.experimental.pallas import tpu_sc as plsc

_K = 16384
_N = 512 * 6 * 4096
_NWORK = 32
_NBKT = 4096
_CAP = 4096
_KCAP = 1024
_MIN32 = np.int32(-2147483648)
_POS = np.int32(2147483647)


def _umap(xi):
    return jnp.where(xi < 0, ~xi, xi | _MIN32)


def _k1_body(x_hbm, hist_hbm, buf, buf2, hist1, outv, sem0, sem1):
    n = x_hbm.shape[0]
    shard = n // _NWORK
    win = 16384
    nwin = shard // win
    nvec_u = win // (16 * 8)

    wid = lax.axis_index("s") * 2 + lax.axis_index("c")
    base = wid * shard

    zeros16 = jnp.zeros((16,), jnp.int32)
    ones16 = jnp.ones((16,), jnp.int32)
    laneoff = lax.iota(jnp.int32, 16) * _NBKT

    @plsc.parallel_loop(0, (16 * _NBKT) // 16, unroll=8)
    def _(i):
        hist1[pl.ds(i * 16, 16)] = zeros16

    bufs = (buf, buf2)
    sems = (sem0, sem1)

    def start(w, b):
        pltpu.async_copy(x_hbm.at[pl.ds(base + w * win, win)], bufs[b], sems[b])

    def wait(w, b):
        pltpu.make_async_copy(
            x_hbm.at[pl.ds(base + w * win, win)], bufs[b], sems[b]).wait()

    start(0, 0)

    def win2_body(w2, _):
        for b in range(2):
            w = w2 * 2 + b

            @pl.when(w + 1 < nwin)
            def _():
                start(w + 1, (b + 1) % 2)

            wait(w, b)
            cur = bufs[b]

            @plsc.parallel_loop(0, win // 16, unroll=8)
            def _(i):
                xi = cur[pl.ds(i * 16, 16)]
                m = _umap(xi)
                bkt = lax.shift_right_logical(m, 20)
                plsc.addupdate_scatter(hist1, [laneoff + bkt], ones16)
        return 0

    lax.fori_loop(0, nwin // 2, win2_body, 0)

    @plsc.parallel_loop(0, _NBKT // 16, unroll=2)
    def _(g):
        acc = hist1[pl.ds(g * 16, 16)]
        for r in range(1, 16):
            acc = acc + hist1[pl.ds(r * _NBKT + g * 16, 16)]
        outv[pl.ds(g * 16, 16)] = acc
    pltpu.sync_copy(outv, hist_hbm.at[wid])


def _k2_body(hist_ref, b_ref):
    merged = jnp.sum(hist_ref[...], axis=0, keepdims=True)
    bidx = lax.broadcasted_iota(jnp.int32, (1, _NBKT), 1)

    def above(b):
        return jnp.sum(jnp.where(bidx > b, merged, 0))

    def bis(_, carry):
        lo, hi = carry
        mid = (lo + hi) // 2
        less = above(mid) < _K
        return (jnp.where(less, lo, mid), jnp.where(less, mid, hi))

    lo0 = jnp.int32(-1)
    hi0 = jnp.int32(_NBKT - 1)
    _, bstar = lax.fori_loop(0, 12, bis, (lo0, hi0))
    b_ref[...] = jnp.full((128,), bstar, jnp.int32)


def _k3_body(x_hbm, b_hbm, cand_hbm, cidx_hbm, cnt_hbm,
             buf, buf2, cand, cidx, bvec, cntv, sem0, sem1):
    n = x_hbm.shape[0]
    shard = n // _NWORK
    win = 32768
    nwin = shard // win
    nvec_u = win // (16 * 4)

    wid = lax.axis_index("s") * 2 + lax.axis_index("c")
    base = wid * shard

    pltpu.sync_copy(b_hbm.at[pl.ds(0, 16)], bvec)
    bv = bvec[...]
    lanes = lax.iota(jnp.int32, 16)

    bufs = (buf, buf2)
    sems = (sem0, sem1)

    def start(w, b):
        pltpu.async_copy(x_hbm.at[pl.ds(base + w * win, win)], bufs[b], sems[b])

    def wait(w, b):
        pltpu.make_async_copy(
            x_hbm.at[pl.ds(base + w * win, win)], bufs[b], sems[b]).wait()

    start(0, 0)

    def win2_body(w2, off):
        for b in range(2):
            w = w2 * 2 + b

            @pl.when(w + 1 < nwin)
            def _():
                start(w + 1, (b + 1) % 2)

            wait(w, b)
            cur = bufs[b]
            wbase = base + w * win

            @plsc.parallel_loop(0, win // 16, unroll=4, carry=off)
            def inner(i, off):
                xi = cur[pl.ds(i * 16, 16)]
                m = _umap(xi)
                bkt = lax.shift_right_logical(m, 20)
                sel = bkt >= bv
                offc = jnp.minimum(off, _CAP - 16)
                plsc.store_compressed(cand.at[pl.ds(offc, 16)], m, mask=sel)
                fidx = (wbase + i * 16) + lanes
                plsc.store_compressed(cidx.at[pl.ds(offc, 16)], fidx, mask=sel)
                return off + plsc.all_reduce_population_count(sel)[0]

            off = inner
        return off

    off = lax.fori_loop(0, nwin // 2, win2_body, jnp.int32(0))
    cnt = jnp.minimum(off, _CAP)

    def cnt_body(t, _):
        cntv[pl.ds(t * 16, 16)] = jnp.full((16,), cnt, jnp.int32)
        return 0

    lax.fori_loop(0, 8, cnt_body, 0)
    pltpu.sync_copy(cand, cand_hbm.at[wid])
    pltpu.sync_copy(cidx, cidx_hbm.at[wid])
    pltpu.sync_copy(cntv, cnt_hbm.at[wid])


def _k4_body(cand_ref, cidx_ref, cnt_ref, t_ref, c_ref):
    cnt0 = cnt_ref[...][:, 0:1]
    valid = lax.broadcasted_iota(jnp.int32, (_NWORK, _CAP), 1) < cnt0
    us = jnp.where(valid, cand_ref[...] ^ _MIN32, _MIN32)

    def count_ge(ts):
        return jnp.sum(jnp.where(us >= ts, 1, 0).astype(jnp.int32))

    def bit_body(k, t_acc):
        bit = lax.shift_left(jnp.int32(1), jnp.int32(31) - k)
        cand_t = t_acc | bit
        ge = count_ge(cand_t ^ _MIN32) >= _K
        return jnp.where(ge, cand_t, t_acc)

    u_t = lax.fori_loop(0, 32, bit_body, jnp.int32(0))
    t_s = u_t ^ _MIN32

    greater = jnp.sum(jnp.where(us > t_s, 1, 0).astype(jnp.int32))
    t_extra = _K - greater
    eqidx = jnp.where(valid & (us == t_s), cidx_ref[...], _POS)

    def count_le(c):
        return jnp.sum(jnp.where(eqidx <= c, 1, 0).astype(jnp.int32))

    def bis_idx(_, carry):
        lo, hi = carry
        mid = (lo + hi) // 2
        ge = count_le(mid) >= t_extra
        return (jnp.where(ge, lo, mid), jnp.where(ge, mid, hi))

    _, cutoff = lax.fori_loop(
        0, 24, bis_idx, (jnp.int32(-1), jnp.int32(_N - 1)))

    t_ref[...] = jnp.full((128,), t_s, jnp.int32)
    c_ref[...] = jnp.full((128,), cutoff, jnp.int32)


def _k6_body(cand_hbm, cidx_hbm, cnt_hbm, t_hbm, c_hbm, out_hbm,
             zbuf, candv, cidxv, vals, idxs, ix0, ix1, ix2, ix3, ix4, ix5,
             ix6, ix7, tv, cv, cntv, zsem, ssem):
    shard = _N // _NWORK
    zwin = 98304
    nz = shard // zwin
    wid = lax.axis_index("s") * 2 + lax.axis_index("c")
    base = wid * shard

    pltpu.sync_copy(t_hbm.at[pl.ds(0, 16)], tv)
    pltpu.sync_copy(c_hbm.at[pl.ds(0, 16)], cv)
    pltpu.sync_copy(cnt_hbm.at[wid], cntv)
    pltpu.sync_copy(cand_hbm.at[wid], candv)
    pltpu.sync_copy(cidx_hbm.at[wid], cidxv)

    ts = tv[...]
    cut = cv[...]
    cntk = cntv[pl.ds(0, 16)]
    lanes = lax.iota(jnp.int32, 16)
    zeros16 = jnp.zeros((16,), jnp.int32)
    minv = jnp.full((16,), _MIN32, jnp.int32)

    @plsc.parallel_loop(0, zwin // 16, unroll=8)
    def _(i):
        zbuf[pl.ds(i * 16, 16)] = zeros16

    zdescs = [
        pltpu.async_copy(zbuf, out_hbm.at[pl.ds(base + w * zwin, zwin)], zsem)
        for w in range(nz)
    ]

    @plsc.parallel_loop(0, _CAP // 16, unroll=4, carry=jnp.int32(0))
    def comp(i, off):
        c = candv[pl.ds(i * 16, 16)]
        ix = cidxv[pl.ds(i * 16, 16)]
        gpos = jnp.full((16,), i * 16, jnp.int32) + lanes
        valid = gpos < cntk
        us = c ^ minv
        keep = valid & ((us > ts) | ((us == ts) & (ix <= cut)))
        xbits = jnp.where(c < 0, c ^ _MIN32, ~c)
        offc = jnp.minimum(off, _KCAP - 16)
        plsc.store_compressed(vals.at[pl.ds(offc, 16)], xbits, mask=keep)
        plsc.store_compressed(idxs.at[pl.ds(offc, 16)], ix, mask=keep)
        return off + plsc.all_reduce_population_count(keep)[0]

    offk = comp

    v0vec = vals[pl.ds(0, 16)]
    i0vec = idxs[pl.ds(0, 16)]
    has = offk > 0
    v0 = jnp.where(has, v0vec[0], 0)
    i0 = jnp.where(has, i0vec[0], base)
    offkv = jnp.full((16,), offk, jnp.int32)
    v0s = jnp.full((16,), v0, jnp.int32)
    i0s = jnp.full((16,), i0, jnp.int32)

    @plsc.parallel_loop(0, _KCAP // 16, unroll=4)
    def pad(g):
        gpos = jnp.full((16,), g * 16, jnp.int32) + lanes
        mask = gpos >= offkv
        vals[pl.ds(g * 16, 16)] = jnp.where(mask, v0s, vals[pl.ds(g * 16, 16)])
        idxs[pl.ds(g * 16, 16)] = jnp.where(mask, i0s, idxs[pl.ds(g * 16, 16)])

    for dsc in zdescs:
        dsc.wait()

    ixrefs = (ix0, ix1, ix2, ix3, ix4, ix5, ix6, ix7)
    for k in range(8):
        for v in range(8):
            ixrefs[k][pl.ds(v * 16, 16)] = idxs[pl.ds(k * 128 + v * 16, 16)]
    sdescs = []
    for k in range(8):
        sdescs.append(
            pltpu.async_copy(vals.at[pl.ds(k * 128, 128)],
                             out_hbm.at[ixrefs[k]], ssem))
    for dsc in sdescs:
        dsc.wait()


def kernel(features):
    b, l, d = features.shape
    n = b * l * d
    flat = features.reshape(n)
    flat_i = lax.optimization_barrier(lax.bitcast_convert_type(flat, jnp.int32))
    mesh = plsc.VectorSubcoreMesh(
        core_axis_name="c", subcore_axis_name="s", num_cores=2, num_subcores=16
    )

    k1 = functools.partial(
        pl.kernel,
        out_type=jax.ShapeDtypeStruct((_NWORK, _NBKT), jnp.int32),
        mesh=mesh,
        scratch_types=[
            pltpu.VMEM((16384,), jnp.int32),
            pltpu.VMEM((16384,), jnp.int32),
            pltpu.VMEM((16 * _NBKT,), jnp.int32),
            pltpu.VMEM((_NBKT,), jnp.int32),
            pltpu.SemaphoreType.DMA,
            pltpu.SemaphoreType.DMA,
        ],
        compiler_params=pltpu.CompilerParams(needs_layout_passes=False),
    )(_k1_body)
    hist = k1(flat_i)

    b_rep = pl.pallas_call(
        _k2_body,
        out_shape=jax.ShapeDtypeStruct((128,), jnp.int32),
    )(hist)

    k3 = functools.partial(
        pl.kernel,
        out_type=(
            jax.ShapeDtypeStruct((_NWORK, _CAP), jnp.int32),
            jax.ShapeDtypeStruct((_NWORK, _CAP), jnp.int32),
            jax.ShapeDtypeStruct((_NWORK, 128), jnp.int32),
        ),
        mesh=mesh,
        scratch_types=[
            pltpu.VMEM((32768,), jnp.int32),
            pltpu.VMEM((32768,), jnp.int32),
            pltpu.VMEM((_CAP,), jnp.int32),
            pltpu.VMEM((_CAP,), jnp.int32),
            pltpu.VMEM((16,), jnp.int32),
            pltpu.VMEM((128,), jnp.int32),
            pltpu.SemaphoreType.DMA,
            pltpu.SemaphoreType.DMA,
        ],
        compiler_params=pltpu.CompilerParams(needs_layout_passes=False),
    )(_k3_body)
    cand, cidx, cnt = k3(flat_i, b_rep)

    tvec, cvec = pl.pallas_call(
        _k4_body,
        in_specs=[
            pl.BlockSpec(memory_space=pltpu.VMEM),
            pl.BlockSpec(memory_space=pltpu.VMEM),
            pl.BlockSpec(memory_space=pltpu.VMEM),
        ],
        out_shape=(
            jax.ShapeDtypeStruct((128,), jnp.int32),
            jax.ShapeDtypeStruct((128,), jnp.int32),
        ),
    )(cand, cidx, cnt)

    k6 = functools.partial(
        pl.kernel,
        out_type=jax.ShapeDtypeStruct((n,), jnp.int32),
        mesh=mesh,
        scratch_types=[
            pltpu.VMEM((98304,), jnp.int32),
            pltpu.VMEM((_CAP,), jnp.int32),
            pltpu.VMEM((_CAP,), jnp.int32),
            pltpu.VMEM((_KCAP,), jnp.int32),
            pltpu.VMEM((_KCAP,), jnp.int32),
        ] + [pltpu.VMEM((128,), jnp.int32) for _ in range(8)] + [
            pltpu.VMEM((16,), jnp.int32),
            pltpu.VMEM((16,), jnp.int32),
            pltpu.VMEM((128,), jnp.int32),
            pltpu.SemaphoreType.DMA,
            pltpu.SemaphoreType.DMA,
        ],
        compiler_params=pltpu.CompilerParams(needs_layout_passes=False),
    )(_k6_body)
    out_bits = k6(cand, cidx, cnt, tvec, cvec)
    out = lax.bitcast_convert_type(out_bits, jnp.float32)
    return out.reshape(b, l, d)


# speedup vs baseline: 1.4414x; 1.4414x over previous
import functools

import jax
import jax.numpy as jnp
import numpy as np
from jax import lax
from jax.experimental import pallas as pl
from jax.experimental.pallas import tpu as pltpu
from jax.experimental.pallas import tpu_sc as plsc

_K = 16384
_N = 512 * 6 * 4096
_NWORK = 32
_NBKT = 2048
_CAP = 4096
_MIN32 = np.int32(-2147483648)
_POS = np.int32(2147483647)


def _umap(xi):
    return jnp.where(xi < 0, ~xi, xi | _MIN32)


def _k1_body(x_hbm, hist_hbm, buf, buf2, hist1, outv, sem0, sem1):
    rows = x_hbm.shape[0]
    rpw = rows // _NWORK
    wr = 8
    nwin = rpw // wr

    wid = lax.axis_index("s") * 2 + lax.axis_index("c")
    rbase = wid * rpw

    zeros16 = jnp.zeros((16,), jnp.int32)
    ones16 = jnp.ones((16,), jnp.int32)
    laneoff = lax.iota(jnp.int32, 16) * _NBKT

    @plsc.parallel_loop(0, (16 * _NBKT) // 16, unroll=8)
    def _(i):
        hist1[pl.ds(i * 16, 16)] = zeros16

    bufs = (buf, buf2)
    sems = (sem0, sem1)

    def start(w, b):
        pltpu.async_copy(
            x_hbm.at[pl.ds(rbase + w * wr, wr), :], bufs[b], sems[b])

    def wait(w, b):
        pltpu.make_async_copy(
            x_hbm.at[pl.ds(rbase + w * wr, wr), :], bufs[b], sems[b]).wait()

    start(0, 0)

    def win2_body(w2, _):
        for b in range(2):
            w = w2 * 2 + b

            @pl.when(w + 1 < nwin)
            def _():
                start(w + 1, (b + 1) % 2)

            wait(w, b)
            cur = bufs[b]
            for r in range(8):
                @plsc.parallel_loop(0, 256, unroll=8)
                def _(i):
                    xi = cur[r, pl.ds(i * 16, 16)]
                    m = _umap(xi)
                    bkt = lax.shift_right_logical(m, 21)
                    plsc.addupdate_scatter(hist1, [laneoff + bkt], ones16)
        return 0

    lax.fori_loop(0, nwin // 2, win2_body, 0)

    @plsc.parallel_loop(0, _NBKT // 16, unroll=2)
    def _(g):
        acc = hist1[pl.ds(g * 16, 16)]
        for r in range(1, 16):
            acc = acc + hist1[pl.ds(r * _NBKT + g * 16, 16)]
        outv[pl.ds(g * 16, 16)] = acc
    pltpu.sync_copy(outv, hist_hbm.at[pl.ds(wid * _NBKT, _NBKT)])


def _k2_body(hist_ref, b_ref, a_ref):
    merged = jnp.sum(hist_ref[...], axis=0, keepdims=True)
    bidx = lax.broadcasted_iota(jnp.int32, (1, _NBKT), 1)

    def above(b):
        return jnp.sum(jnp.where(bidx > b, merged, 0))

    def bis(_, carry):
        lo, hi = carry
        mid = (lo + hi) // 2
        less = above(mid) < _K
        return (jnp.where(less, lo, mid), jnp.where(less, mid, hi))

    lo0 = jnp.int32(-1)
    hi0 = jnp.int32(_NBKT - 1)
    _, bstar = lax.fori_loop(0, 12, bis, (lo0, hi0))
    a = above(bstar)
    b_ref[...] = jnp.full((128,), bstar, jnp.int32)
    a_ref[...] = jnp.full((128,), a, jnp.int32)


def _k3_body(x_hbm, b_hbm, cand_hbm, cidx_hbm, cnt_hbm,
             buf, buf2, cand, cidx, bvec, cntv, sem0, sem1):
    rows = x_hbm.shape[0]
    rpw = rows // _NWORK
    wr = 8
    nwin = rpw // wr

    wid = lax.axis_index("s") * 2 + lax.axis_index("c")
    rbase = wid * rpw

    pltpu.sync_copy(b_hbm.at[pl.ds(0, 16)], bvec)
    bv = bvec[...]
    lanes = lax.iota(jnp.int32, 16)

    bufs = (buf, buf2)
    sems = (sem0, sem1)

    def start(w, b):
        pltpu.async_copy(
            x_hbm.at[pl.ds(rbase + w * wr, wr), :], bufs[b], sems[b])

    def wait(w, b):
        pltpu.make_async_copy(
            x_hbm.at[pl.ds(rbase + w * wr, wr), :], bufs[b], sems[b]).wait()

    start(0, 0)

    def win2_body(w2, off):
        for b in range(2):
            w = w2 * 2 + b

            @pl.when(w + 1 < nwin)
            def _():
                start(w + 1, (b + 1) % 2)

            wait(w, b)
            cur = bufs[b]
            for r in range(8):
                rowidx = (rbase + w * wr + r) * 4096

                @plsc.parallel_loop(0, 256, unroll=4, carry=off)
                def inner(i, off):
                    xi = cur[r, pl.ds(i * 16, 16)]
                    m = _umap(xi)
                    bkt = lax.shift_right_logical(m, 21)
                    sel = bkt == bv
                    offc = jnp.minimum(off, _CAP - 16)
                    plsc.store_compressed(cand.at[pl.ds(offc, 16)], m,
                                          mask=sel)
                    fidx = (rowidx + i * 16) + lanes
                    plsc.store_compressed(cidx.at[pl.ds(offc, 16)], fidx,
                                          mask=sel)
                    return off + plsc.all_reduce_population_count(sel)[0]

                off = inner
        return off

    off = lax.fori_loop(0, nwin // 2, win2_body, jnp.int32(0))
    cnt = jnp.minimum(off, _CAP)

    def cnt_body(t, _):
        cntv[pl.ds(t * 16, 16)] = jnp.full((16,), cnt, jnp.int32)
        return 0

    lax.fori_loop(0, 8, cnt_body, 0)
    pltpu.sync_copy(cand, cand_hbm.at[pl.ds(wid * _CAP, _CAP)])
    pltpu.sync_copy(cidx, cidx_hbm.at[pl.ds(wid * _CAP, _CAP)])
    pltpu.sync_copy(cntv, cnt_hbm.at[pl.ds(wid * 128, 128)])


def _k4_body(b_smem, a_smem, cand_ref, cidx_ref, cnt_ref, t_ref, c_ref):
    bstar = b_smem[0]
    above = a_smem[0]
    j = _K - above

    cnt0 = cnt_ref[...][:, 0:1]
    valid = lax.broadcasted_iota(jnp.int32, (_NWORK, _CAP), 1) < cnt0
    low = jnp.where(valid, cand_ref[...] & 0x1FFFFF, -1)

    def count_ge(t):
        return jnp.sum(jnp.where(low >= t, 1, 0).astype(jnp.int32))

    def bis(_, carry):
        lo, hi = carry
        mid = (lo + hi) // 2
        ge = count_ge(mid) >= j
        return (jnp.where(ge, mid, lo), jnp.where(ge, hi, mid))

    tlow, _ = lax.fori_loop(0, 21, bis, (jnp.int32(0), jnp.int32(1 << 21)))

    greater = above + count_ge(tlow + 1)
    t_extra = _K - greater
    eqidx = jnp.where(valid & (low == tlow), cidx_ref[...], _POS)

    def count_le(c):
        return jnp.sum(jnp.where(eqidx <= c, 1, 0).astype(jnp.int32))

    def bis_idx(_, carry):
        lo, hi = carry
        mid = (lo + hi) // 2
        ge = count_le(mid) >= t_extra
        return (jnp.where(ge, lo, mid), jnp.where(ge, mid, hi))

    _, cutoff = lax.fori_loop(
        0, 24, bis_idx, (jnp.int32(-1), jnp.int32(_N - 1)))

    u_t = (bstar << 21) | tlow
    t_ref[...] = jnp.full((128,), u_t ^ _MIN32, jnp.int32)
    c_ref[...] = jnp.full((128,), cutoff, jnp.int32)


def _k5_body(t_smem, c_smem, x_ref, o_ref):
    ts = t_smem[0]
    cutoff = c_smem[0]
    x = x_ref[...]
    rows, d = x_ref.shape
    xi = pltpu.bitcast(x, jnp.int32)
    us = jnp.where(xi < 0, xi ^ _POS, xi)
    base = pl.program_id(0) * (rows * d)
    fidx = (base
            + lax.broadcasted_iota(jnp.int32, (rows, d), 0) * d
            + lax.broadcasted_iota(jnp.int32, (rows, d), 1))
    keep = (us > ts) | ((us == ts) & (fidx <= cutoff))
    o_ref[...] = jnp.where(keep, x, jnp.float32(0.0))


def kernel(features):
    b, l, d = features.shape
    n = b * l * d
    rows = b * l
    flat2 = features.reshape(rows, d)
    x2i = lax.bitcast_convert_type(flat2, jnp.int32)
    mesh = plsc.VectorSubcoreMesh(
        core_axis_name="c", subcore_axis_name="s", num_cores=2, num_subcores=16
    )
    sc_params = pltpu.CompilerParams(
        needs_layout_passes=False, use_tc_tiling_on_sc=True)

    k1 = functools.partial(
        pl.kernel,
        out_type=jax.ShapeDtypeStruct((_NWORK * _NBKT,), jnp.int32),
        mesh=mesh,
        scratch_types=[
            pltpu.VMEM((8, 4096), jnp.int32),
            pltpu.VMEM((8, 4096), jnp.int32),
            pltpu.VMEM((16 * _NBKT,), jnp.int32),
            pltpu.VMEM((_NBKT,), jnp.int32),
            pltpu.SemaphoreType.DMA,
            pltpu.SemaphoreType.DMA,
        ],
        compiler_params=sc_params,
    )(_k1_body)
    hist = k1(x2i)

    b_rep, a_rep = pl.pallas_call(
        _k2_body,
        out_shape=(
            jax.ShapeDtypeStruct((128,), jnp.int32),
            jax.ShapeDtypeStruct((128,), jnp.int32),
        ),
    )(hist.reshape(_NWORK, _NBKT))

    k3 = functools.partial(
        pl.kernel,
        out_type=(
            jax.ShapeDtypeStruct((_NWORK * _CAP,), jnp.int32),
            jax.ShapeDtypeStruct((_NWORK * _CAP,), jnp.int32),
            jax.ShapeDtypeStruct((_NWORK * 128,), jnp.int32),
        ),
        mesh=mesh,
        scratch_types=[
            pltpu.VMEM((8, 4096), jnp.int32),
            pltpu.VMEM((8, 4096), jnp.int32),
            pltpu.VMEM((_CAP,), jnp.int32),
            pltpu.VMEM((_CAP,), jnp.int32),
            pltpu.VMEM((16,), jnp.int32),
            pltpu.VMEM((128,), jnp.int32),
            pltpu.SemaphoreType.DMA,
            pltpu.SemaphoreType.DMA,
        ],
        compiler_params=sc_params,
    )(_k3_body)
    cand, cidx, cnt = k3(x2i, b_rep)

    tvec, cvec = pl.pallas_call(
        _k4_body,
        in_specs=[
            pl.BlockSpec(memory_space=pltpu.SMEM),
            pl.BlockSpec(memory_space=pltpu.SMEM),
            pl.BlockSpec(memory_space=pltpu.VMEM),
            pl.BlockSpec(memory_space=pltpu.VMEM),
            pl.BlockSpec(memory_space=pltpu.VMEM),
        ],
        out_shape=(
            jax.ShapeDtypeStruct((128,), jnp.int32),
            jax.ShapeDtypeStruct((128,), jnp.int32),
        ),
    )(b_rep, a_rep, cand.reshape(_NWORK, _CAP), cidx.reshape(_NWORK, _CAP),
      cnt.reshape(_NWORK, 128))

    blk = 256
    out = pl.pallas_call(
        _k5_body,
        grid=(rows // blk,),
        in_specs=[
            pl.BlockSpec(memory_space=pltpu.SMEM),
            pl.BlockSpec(memory_space=pltpu.SMEM),
            pl.BlockSpec((blk, d), lambda i: (i, 0)),
        ],
        out_specs=pl.BlockSpec((blk, d), lambda i: (i, 0)),
        out_shape=jax.ShapeDtypeStruct((rows, d), jnp.float32),
    )(tvec, cvec, flat2)
    return out.reshape(b, l, d)
